# baseline (speedup 1.0000x reference)
import jax
import jax.numpy as jnp
from jax.experimental import pallas as pl
from jax.experimental.pallas import tpu as pltpu

_NOISE = 0.1
_ROWS = 192 * 384 * 384 // 384
_W = 384
_B = 1024
_N = _ROWS // _B


def _sp_kernel(img_ref, noise_ref, out_ref, acc_ref):
    i = pl.program_id(0)

    @pl.when(i == 0)
    def _init():
        acc_ref[0] = jnp.inf
        acc_ref[1] = -jnp.inf

    @pl.when(i < _N)
    def _reduce():
        x = img_ref[...]
        acc_ref[0] = jnp.minimum(acc_ref[0], jnp.min(x))
        acc_ref[1] = jnp.maximum(acc_ref[1], jnp.max(x))

    @pl.when(i >= _N)
    def _apply():
        x = img_ref[...]
        u = noise_ref[...]
        mn = acc_ref[0]
        mx = acc_ref[1]
        out = jnp.where(u < _NOISE, mn, x)
        out_ref[...] = jnp.where(u > 1.0 - _NOISE, mx, out)


def kernel(img, noise_u):
    x = img.reshape(_ROWS, _W)
    u = noise_u.reshape(_ROWS, _W)
    out = pl.pallas_call(
        _sp_kernel,
        grid=(2 * _N,),
        in_specs=[
            pl.BlockSpec((_B, _W), lambda i: (i % _N, 0)),
            pl.BlockSpec((_B, _W), lambda i: (jnp.where(i < _N, 0, i - _N), 0)),
        ],
        out_specs=pl.BlockSpec((_B, _W), lambda i: (jnp.where(i < _N, 0, i - _N), 0)),
        out_shape=jax.ShapeDtypeStruct((_ROWS, _W), jnp.float32),
        scratch_shapes=[pltpu.SMEM((2,), jnp.float32)],
    )(x, u)
    return out.reshape(img.shape)


# speedup vs baseline: 1.3454x; 1.3454x over previous
import jax
import jax.numpy as jnp
from jax.experimental import pallas as pl
from jax.experimental.pallas import tpu as pltpu

_NOISE = 0.1
_ROWS = 192 * 384 * 384 // 384
_W = 384
_B = 4096
_N = _ROWS // _B


def _sp_kernel(img_ref, noise_ref, out_ref, vmin_ref, vmax_ref):
    i = pl.program_id(0)

    @pl.when(i == 0)
    def _init():
        vmin_ref[...] = jnp.full((8, _W), jnp.inf, jnp.float32)
        vmax_ref[...] = jnp.full((8, _W), -jnp.inf, jnp.float32)

    @pl.when(i < _N)
    def _reduce():
        x = img_ref[...].reshape(_B // 8, 8, _W)
        vmin_ref[...] = jnp.minimum(vmin_ref[...], jnp.min(x, axis=0))
        vmax_ref[...] = jnp.maximum(vmax_ref[...], jnp.max(x, axis=0))

    @pl.when(i >= _N)
    def _apply():
        x = img_ref[...]
        u = noise_ref[...]
        mn = jnp.min(vmin_ref[...])
        mx = jnp.max(vmax_ref[...])
        out = jnp.where(u < _NOISE, mn, x)
        out_ref[...] = jnp.where(u > 1.0 - _NOISE, mx, out)


def kernel(img, noise_u):
    x = img.reshape(_ROWS, _W)
    u = noise_u.reshape(_ROWS, _W)
    out = pl.pallas_call(
        _sp_kernel,
        grid=(2 * _N,),
        in_specs=[
            pl.BlockSpec((_B, _W), lambda i: (i % _N, 0)),
            pl.BlockSpec((_B, _W), lambda i: (jnp.where(i < _N, 0, i - _N), 0)),
        ],
        out_specs=pl.BlockSpec((_B, _W), lambda i: (jnp.where(i < _N, 0, i - _N), 0)),
        out_shape=jax.ShapeDtypeStruct((_ROWS, _W), jnp.float32),
        scratch_shapes=[
            pltpu.VMEM((8, _W), jnp.float32),
            pltpu.VMEM((8, _W), jnp.float32),
        ],
    )(x, u)
    return out.reshape(img.shape)
